# Initial kernel scaffold; baseline (speedup 1.0000x reference)
import jax
import jax.numpy as jnp
from jax import lax
from jax.experimental import pallas as pl
from jax.experimental.pallas import tpu as pltpu
from jax.experimental.pallas import tpu_sc as plsc

B = 4096
N_FIELDS = 26
VOCAB = 100000
D_EMB = 64

NUM_CORES = 2
NUM_SUBCORES = 16
NUM_WORKERS = NUM_CORES * NUM_SUBCORES
B_PER_W = B // NUM_WORKERS


def _sc_gather_sum(idx, tables_flat):
    mesh = plsc.VectorSubcoreMesh(
        core_axis_name="c", subcore_axis_name="s",
        num_cores=NUM_CORES, num_subcores=NUM_SUBCORES,
    )

    def body(idx_hbm, tab_hbm, s_hbm, idx_v, acc_v, sem):
        cid = lax.axis_index("c")
        sid = lax.axis_index("s")
        wid = sid * NUM_CORES + cid
        pltpu.sync_copy(idx_hbm.at[wid], idx_v)
        pltpu.async_copy(tab_hbm.at[idx_v.at[0]], acc_v, sem).wait()
        pending = []
        for f in range(1, N_FIELDS):
            pending.append(
                pltpu.async_copy(tab_hbm.at[idx_v.at[f]], acc_v, sem, add=True))
            if f in (13, N_FIELDS - 1):
                for cp in pending:
                    cp.wait()
                pending = []
        pltpu.sync_copy(acc_v, s_hbm.at[pl.ds(wid * B_PER_W, B_PER_W)])

    call = pl.kernel(
        body,
        out_type=jax.ShapeDtypeStruct((B, D_EMB), jnp.float32),
        mesh=mesh,
        scratch_types=[
            pltpu.VMEM((N_FIELDS, B_PER_W), jnp.int32),
            pltpu.VMEM((B_PER_W, D_EMB), jnp.float32),
            pltpu.SemaphoreType.DMA,
        ],
    )
    return call(idx, tables_flat)


def _tc_mlp(s, W1, b1, W2, b2):
    def body(s_ref, w1_ref, b1_ref, w2_ref, b2_ref, o_ref):
        h = jnp.dot(s_ref[...], w1_ref[...].T,
                    preferred_element_type=jnp.float32)
        h = h + jnp.float32(N_FIELDS) * b1_ref[...]
        o = jnp.dot(h, w2_ref[...].T, preferred_element_type=jnp.float32)
        o_ref[...] = o + jnp.float32(N_FIELDS) * b2_ref[...]

    return pl.pallas_call(
        body,
        out_shape=jax.ShapeDtypeStruct((B, 1), jnp.float32),
    )(s, W1, b1.reshape(1, 32), W2, b2.reshape(1, 1))


def kernel(src, tables, W1, b1, W2, b2):
    src = src.astype(jnp.int32)
    offs = (jnp.arange(N_FIELDS, dtype=jnp.int32) * VOCAB)[None, :]
    idx = (src + offs).reshape(NUM_WORKERS, B_PER_W, N_FIELDS)
    idx = idx.transpose(0, 2, 1)
    tables_flat = tables.reshape(N_FIELDS * VOCAB, D_EMB)
    s = _sc_gather_sum(idx, tables_flat)
    return _tc_mlp(s, W1, b1, W2, b2)


# speedup vs baseline: 1.1178x; 1.1178x over previous
import jax
import jax.numpy as jnp
from jax import lax
from jax.experimental import pallas as pl
from jax.experimental.pallas import tpu as pltpu
from jax.experimental.pallas import tpu_sc as plsc

B = 4096
N_FIELDS = 26
VOCAB = 100000
D_EMB = 64

NUM_CORES = 2
NUM_SUBCORES = 16
NUM_WORKERS = NUM_CORES * NUM_SUBCORES
B_PER_W = B // NUM_WORKERS


def _sc_gather_sum(idx, tables_flat):
    mesh = plsc.VectorSubcoreMesh(
        core_axis_name="c", subcore_axis_name="s",
        num_cores=NUM_CORES, num_subcores=NUM_SUBCORES,
    )

    def body(idx_hbm, tab_hbm, s_hbm, idx_v, acc_v, sem):
        cid = lax.axis_index("c")
        sid = lax.axis_index("s")
        wid = sid * NUM_CORES + cid
        pltpu.sync_copy(idx_hbm.at[wid], idx_v)
        pltpu.async_copy(tab_hbm.at[idx_v.at[0]], acc_v, sem).wait()
        pending = []
        for f in range(1, N_FIELDS):
            pending.append(
                pltpu.async_copy(tab_hbm.at[idx_v.at[f]], acc_v, sem, add=True))
            if f in (13, N_FIELDS - 1):
                for cp in pending:
                    cp.wait()
                pending = []
        pltpu.sync_copy(acc_v, s_hbm.at[pl.ds(wid * B_PER_W, B_PER_W)])

    call = pl.kernel(
        body,
        out_type=jax.ShapeDtypeStruct((B, D_EMB), jnp.float32),
        mesh=mesh,
        scratch_types=[
            pltpu.VMEM((N_FIELDS, B_PER_W), jnp.int32),
            pltpu.VMEM((B_PER_W, D_EMB), jnp.float32),
            pltpu.SemaphoreType.DMA,
        ],
        compiler_params=pltpu.CompilerParams(use_tc_tiling_on_sc=False),
    )
    return call(idx, tables_flat)


def _tc_mlp(s, W1, b1, W2, b2):
    def body(s_ref, w1_ref, b1_ref, w2_ref, b2_ref, o_ref):
        h = jnp.dot(s_ref[...], w1_ref[...].T,
                    preferred_element_type=jnp.float32)
        h = h + jnp.float32(N_FIELDS) * b1_ref[...]
        o = jnp.dot(h, w2_ref[...], preferred_element_type=jnp.float32)
        o_ref[...] = o + jnp.float32(N_FIELDS) * b2_ref[0]

    w2p = jnp.zeros((32, 128), jnp.float32).at[:, 0].set(W2[0])
    out = pl.pallas_call(
        body,
        in_specs=[
            pl.BlockSpec(memory_space=pltpu.VMEM),
            pl.BlockSpec(memory_space=pltpu.VMEM),
            pl.BlockSpec(memory_space=pltpu.VMEM),
            pl.BlockSpec(memory_space=pltpu.VMEM),
            pl.BlockSpec(memory_space=pltpu.SMEM),
        ],
        out_shape=jax.ShapeDtypeStruct((B, 128), jnp.float32),
    )(s, W1, b1.reshape(1, 32), w2p, b2.reshape(1,))
    return out[:, :1]


def kernel(src, tables, W1, b1, W2, b2):
    src = src.astype(jnp.int32)
    offs = (jnp.arange(N_FIELDS, dtype=jnp.int32) * VOCAB)[None, :]
    idx = (src + offs).reshape(NUM_WORKERS, B_PER_W, N_FIELDS)
    idx = idx.transpose(0, 2, 1)
    tables_flat = tables.reshape(N_FIELDS * VOCAB, D_EMB)
    s = _sc_gather_sum(idx, tables_flat)
    return _tc_mlp(s, W1, b1, W2, b2)


# speedup vs baseline: 3.0286x; 2.7096x over previous
import jax
import jax.numpy as jnp
from jax import lax
from jax.experimental import pallas as pl
from jax.experimental.pallas import tpu as pltpu
from jax.experimental.pallas import tpu_sc as plsc

B = 4096
N_FIELDS = 26
VOCAB = 100000
D_EMB = 64

NUM_CORES = 2
NUM_SUBCORES = 16
NUM_WORKERS = NUM_CORES * NUM_SUBCORES
B_PER_W = B // NUM_WORKERS

ROWS_PER_TILE = 8
N_TILES = N_FIELDS * VOCAB // ROWS_PER_TILE
B_PER_CHUNK = 4
PAIRS_PER_CHUNK = B_PER_CHUNK * N_FIELDS
N_CHUNKS = B_PER_W // B_PER_CHUNK
PAIRS_PER_W = B_PER_W * N_FIELDS
LANES = 16
CGROUPS = D_EMB // LANES


def _sc_gather_sum(tile_ids, row_ids, tiles4):
    mesh = plsc.VectorSubcoreMesh(
        core_axis_name="c", subcore_axis_name="s",
        num_cores=NUM_CORES, num_subcores=NUM_SUBCORES,
    )

    def body(tid_hbm, rid_hbm, tab_hbm, s_hbm, tid_v, rid_v, out_v, rows_v,
             gat_sem):
        cid = lax.axis_index("c")
        sid = lax.axis_index("s")
        wid = sid * NUM_CORES + cid
        pbase = wid * PAIRS_PER_W
        pltpu.sync_copy(tid_hbm.at[pl.ds(pbase, PAIRS_PER_W)],
                        tid_v.at[pl.ds(0, PAIRS_PER_W)])
        pltpu.sync_copy(rid_hbm.at[pl.ds(pbase, PAIRS_PER_W)],
                        rid_v.at[pl.ds(0, PAIRS_PER_W)])

        n_vec = (PAIRS_PER_CHUNK + LANES - 1) // LANES

        def chunk_body(c, carry):
            base = c * PAIRS_PER_CHUNK
            tvs = [tid_v[pl.ds(base + k * LANES, LANES)] for k in range(n_vec)]
            rvs = [rid_v[pl.ds(base + k * LANES, LANES)] for k in range(n_vec)]
            cps = []
            for i in range(PAIRS_PER_CHUNK):
                t = tvs[i // LANES][i % LANES]
                r = rvs[i // LANES][i % LANES]
                cps.append(pltpu.async_copy(
                    tab_hbm.at[t, pl.ds(r, 1)],
                    rows_v.at[pl.ds(i, 1)], gat_sem))
            for cp in cps:
                cp.wait()
            for bl in range(B_PER_CHUNK):
                accs = [jnp.zeros((LANES,), jnp.float32)
                        for _ in range(CGROUPS)]
                for f in range(N_FIELDS):
                    i = bl * N_FIELDS + f
                    for g in range(CGROUPS):
                        accs[g] = accs[g] + rows_v[i,
                                                   pl.ds(g * LANES, LANES)]
                ob = (c * B_PER_CHUNK + bl) * D_EMB
                for g in range(CGROUPS):
                    out_v[pl.ds(ob + g * LANES, LANES)] = accs[g]
            return carry

        lax.fori_loop(0, N_CHUNKS, chunk_body, 0)
        pltpu.sync_copy(out_v, s_hbm.at[pl.ds(wid * B_PER_W * D_EMB,
                                              B_PER_W * D_EMB)])

    call = pl.kernel(
        body,
        out_type=jax.ShapeDtypeStruct((B * D_EMB,), jnp.float32),
        mesh=mesh,
        name="sc_gather_sum",
        scratch_types=[
            pltpu.VMEM((PAIRS_PER_W + LANES,), jnp.int32),
            pltpu.VMEM((PAIRS_PER_W + LANES,), jnp.int32),
            pltpu.VMEM((B_PER_W * D_EMB,), jnp.float32),
            pltpu.VMEM((PAIRS_PER_CHUNK, D_EMB), jnp.float32),
            pltpu.SemaphoreType.DMA,
        ],
        compiler_params=pltpu.CompilerParams(use_tc_tiling_on_sc=True),
    )
    return call(tile_ids, row_ids, tiles4)


def _tc_mlp(s, W1, b1, W2, b2):
    def body(s_ref, w1_ref, b1_ref, w2_ref, b2_ref, o_ref):
        h = jnp.dot(s_ref[...], w1_ref[...].T,
                    preferred_element_type=jnp.float32)
        h = h + jnp.float32(N_FIELDS) * b1_ref[...]
        o = jnp.dot(h, w2_ref[...], preferred_element_type=jnp.float32)
        o_ref[...] = o + jnp.float32(N_FIELDS) * b2_ref[0]

    w2p = jnp.zeros((32, 128), jnp.float32).at[:, 0].set(W2[0])
    out = pl.pallas_call(
        body,
        in_specs=[
            pl.BlockSpec(memory_space=pltpu.VMEM),
            pl.BlockSpec(memory_space=pltpu.VMEM),
            pl.BlockSpec(memory_space=pltpu.VMEM),
            pl.BlockSpec(memory_space=pltpu.VMEM),
            pl.BlockSpec(memory_space=pltpu.SMEM),
        ],
        out_shape=jax.ShapeDtypeStruct((B, 128), jnp.float32),
    )(s, W1, b1.reshape(1, 32), w2p, b2.reshape(1,))
    return out[:, :1]


def kernel(src, tables, W1, b1, W2, b2):
    src = src.astype(jnp.int32)
    offs = (jnp.arange(N_FIELDS, dtype=jnp.int32) * VOCAB)[None, :]
    flat = (src + offs).reshape(-1)
    tile_ids = flat >> 3
    row_ids = flat & 7
    tiles4 = tables.reshape(N_TILES, ROWS_PER_TILE, D_EMB)
    s = _sc_gather_sum(tile_ids, row_ids, tiles4)
    return _tc_mlp(s.reshape(B, D_EMB), W1, b1, W2, b2)
